# baseline (speedup 1.0000x reference)
import functools

import jax
import jax.numpy as jnp
from jax import lax
from jax.experimental import pallas as pl
from jax.experimental.pallas import tpu as pltpu
from jax.experimental.pallas import tpu_sc as plsc

RES = 256
SIDE = 257
F = 4
M = SIDE * SIDE * SIDE
NC, NS, L = 2, 16, 16
NW = NC * NS

C = 1024
G = C // L

_OFFS = (0, 1, SIDE, SIDE + 1, SIDE * SIDE, SIDE * SIDE + 1,
         SIDE * SIDE + SIDE, SIDE * SIDE + SIDE + 1)


def _body(chunks_pw, npad, xT, table, out, xv, idxbuf, wbuf, rows, outbuf,
          sem):
    wid = lax.axis_index("s") * NC + lax.axis_index("c")
    wbase = wid * (chunks_pw * C)

    def chunk_body(g, _):
        base = wbase + g * C
        pltpu.sync_copy(xT.at[:, pl.ds(base, C)], xv)

        def pass1(i, _):
            p = i * L
            idv = []
            for d in range(3):
                xs = xv[d, pl.ds(p, L)] * float(RES)
                idx = xs.astype(jnp.int32)
                lo = xs - idx.astype(jnp.float32)
                wbuf[d, pl.ds(p, L)] = (3.0 - 2.0 * lo) * lo * lo
                idv.append(idx)
            b4 = (idv[0] * (SIDE * SIDE) + idv[1] * SIDE + idv[2]) * F
            for c in range(8):
                for f in range(F):
                    idxbuf[pl.ds((c * F + f) * C + p, L)] = (
                        b4 + (_OFFS[c] * F + f))
            return 0

        lax.fori_loop(0, G, pass1, 0)
        pltpu.async_copy(table.at[idxbuf], rows, sem).wait()

        def pass2(i, _):
            p = i * L
            w0 = wbuf[0, pl.ds(p, L)]
            w1 = wbuf[1, pl.ds(p, L)]
            w2 = wbuf[2, pl.ds(p, L)]
            for f in range(F):
                v = [rows[pl.ds((c * F + f) * C + p, L)] for c in range(8)]
                m00 = v[0] + w2 * (v[1] - v[0])
                m01 = v[2] + w2 * (v[3] - v[2])
                m10 = v[4] + w2 * (v[5] - v[4])
                m11 = v[6] + w2 * (v[7] - v[6])
                n0 = m00 + w1 * (m01 - m00)
                n1 = m10 + w1 * (m11 - m10)
                outbuf[f, pl.ds(p, L)] = n0 + w0 * (n1 - n0)
            return 0

        lax.fori_loop(0, G, pass2, 0)
        for f in range(F):
            pltpu.sync_copy(outbuf.at[f, :], out.at[pl.ds(f * npad + base, C)])
        return 0

    lax.fori_loop(0, chunks_pw, chunk_body, 0)


@functools.partial(jax.jit, static_argnums=(2,))
def _run(xT, table, npad):
    mesh = plsc.VectorSubcoreMesh(core_axis_name="c", subcore_axis_name="s")
    chunks_pw = npad // (NW * C)
    kfn = pl.kernel(
        functools.partial(_body, chunks_pw, npad),
        out_type=jax.ShapeDtypeStruct((F * npad,), jnp.float32),
        mesh=mesh,
        scratch_types=[
            pltpu.VMEM((3, C), jnp.float32),
            pltpu.VMEM((8 * F * C,), jnp.int32),
            pltpu.VMEM((3, C), jnp.float32),
            pltpu.VMEM((8 * F * C,), jnp.float32),
            pltpu.VMEM((F, C), jnp.float32),
            pltpu.SemaphoreType.DMA,
        ],
    )
    return kfn(xT, table)


def kernel(x, values):
    n = x.shape[0]
    step = NW * C
    npad = ((n + step - 1) // step) * step
    xp = jnp.pad(x, ((0, npad - n), (0, 0)))
    out = _run(xp.T, values.reshape(M * F), npad)
    return out.reshape(F, npad)[:, :n].T


# speedup vs baseline: 1.0477x; 1.0477x over previous
import functools

import jax
import jax.numpy as jnp
from jax import lax
from jax.experimental import pallas as pl
from jax.experimental.pallas import tpu as pltpu
from jax.experimental.pallas import tpu_sc as plsc

RES = 256
SIDE = 257
F = 4
M = SIDE * SIDE * SIDE
NC, NS, L = 2, 16, 16
NW = NC * NS

C = 512
G = C // L
S = 8
SZ = 8 * F * C // S

_OFFS = (0, 1, SIDE, SIDE + 1, SIDE * SIDE, SIDE * SIDE + 1,
         SIDE * SIDE + SIDE, SIDE * SIDE + SIDE + 1)


def _body(chunks_pw, npad, xT, table, out,
          xv0, idx0, w0, rows0, ob0, sem0,
          xv1, idx1, w1, rows1, ob1, sem1):
    wid = lax.axis_index("s") * NC + lax.axis_index("c")
    wbase = wid * (chunks_pw * C)
    bufs = ((xv0, idx0, w0, rows0, ob0, sem0),
            (xv1, idx1, w1, rows1, ob1, sem1))

    def fire(g, xv, idxbuf, wbuf, rows, sem):
        base = wbase + g * C
        pltpu.sync_copy(xT.at[:, pl.ds(base, C)], xv)

        def pass1(i, _):
            p = i * L
            idv = []
            for d in range(3):
                xs = xv[d, pl.ds(p, L)] * float(RES)
                idx = xs.astype(jnp.int32)
                lo = xs - idx.astype(jnp.float32)
                wbuf[d, pl.ds(p, L)] = (3.0 - 2.0 * lo) * lo * lo
                idv.append(idx)
            b4 = (idv[0] * (SIDE * SIDE) + idv[1] * SIDE + idv[2]) * F
            for c in range(8):
                for f in range(F):
                    idxbuf[pl.ds((c * F + f) * C + p, L)] = (
                        b4 + (_OFFS[c] * F + f))
            return 0

        lax.fori_loop(0, G, pass1, 0)
        for s in range(S):
            pltpu.async_copy(table.at[idxbuf.at[pl.ds(s * SZ, SZ)]],
                             rows.at[pl.ds(s * SZ, SZ)], sem)

    def drain(g, wbuf, rows, outbuf, sem):
        base = wbase + g * C
        pltpu.make_async_copy(table.at[pl.ds(0, 8 * F * C)], rows, sem).wait()

        def pass2(i, _):
            p = i * L
            w_0 = wbuf[0, pl.ds(p, L)]
            w_1 = wbuf[1, pl.ds(p, L)]
            w_2 = wbuf[2, pl.ds(p, L)]
            for f in range(F):
                v = [rows[pl.ds((c * F + f) * C + p, L)] for c in range(8)]
                m00 = v[0] + w_2 * (v[1] - v[0])
                m01 = v[2] + w_2 * (v[3] - v[2])
                m10 = v[4] + w_2 * (v[5] - v[4])
                m11 = v[6] + w_2 * (v[7] - v[6])
                n0 = m00 + w_1 * (m01 - m00)
                n1 = m10 + w_1 * (m11 - m10)
                outbuf[f, pl.ds(p, L)] = n0 + w_0 * (n1 - n0)
            return 0

        lax.fori_loop(0, G, pass2, 0)
        for f in range(F):
            pltpu.sync_copy(outbuf.at[f, :], out.at[pl.ds(f * npad + base, C)])

    def fire_b(g, b):
        xv, idxbuf, wbuf, rows, _, sem = bufs[b]
        fire(g, xv, idxbuf, wbuf, rows, sem)

    def drain_b(g, b):
        _, _, wbuf, rows, outbuf, sem = bufs[b]
        drain(g, wbuf, rows, outbuf, sem)

    half = chunks_pw // 2
    fire_b(wbase * 0, 0)

    def body2(t, _):
        g0 = 2 * t
        fire_b(g0 + 1, 1)
        drain_b(g0, 0)

        @pl.when(t + 1 < half)
        def _():
            fire_b(g0 + 2, 0)

        drain_b(g0 + 1, 1)
        return 0

    lax.fori_loop(0, half, body2, 0)


@functools.partial(jax.jit, static_argnums=(2,))
def _run(xT, table, npad):
    mesh = plsc.VectorSubcoreMesh(core_axis_name="c", subcore_axis_name="s")
    chunks_pw = npad // (NW * C)
    buf = [
        pltpu.VMEM((3, C), jnp.float32),
        pltpu.VMEM((8 * F * C,), jnp.int32),
        pltpu.VMEM((3, C), jnp.float32),
        pltpu.VMEM((8 * F * C,), jnp.float32),
        pltpu.VMEM((F, C), jnp.float32),
        pltpu.SemaphoreType.DMA,
    ]
    kfn = pl.kernel(
        functools.partial(_body, chunks_pw, npad),
        out_type=jax.ShapeDtypeStruct((F * npad,), jnp.float32),
        mesh=mesh,
        scratch_types=buf + buf,
    )
    return kfn(xT, table)


def kernel(x, values):
    n = x.shape[0]
    step = 2 * NW * C
    npad = ((n + step - 1) // step) * step
    xp = jnp.pad(x, ((0, npad - n), (0, 0)))
    out = _run(xp.T, values.reshape(M * F), npad)
    return out.reshape(F, npad)[:, :n].T
